# baseline (speedup 1.0000x reference)
import functools

import jax
import jax.numpy as jnp
from jax import lax
from jax.experimental import pallas as pl
from jax.experimental.pallas import tpu as pltpu
from jax.experimental.pallas import tpu_sc as plsc

N = 8_000_000
NG = 4
NBINS = 51
NQ = NBINS - 1
NC = 2
NS = 16
NW = NC * NS
PERW = N // NW
CH = 10_000
NCHUNK = PERW // CH
CH3 = 5_952
NCHUNK3 = PERW // CH3
CH3T = PERW - NCHUNK3 * CH3

H1 = NG * 65536
H1P = H1 + 128
H1S = H1P // NS
S2 = NG * 128
H2 = S2 * 256
H2P = H2 + 128
H2S = H2P // NS
BIG = 2**31 - 1

@functools.cache
def _mesh():
    return plsc.VectorSubcoreMesh(
        core_axis_name="c", subcore_axis_name="s",
        num_cores=NC, num_subcores=NS)


def _key_u32(xk):
    b = lax.bitcast_convert_type(xk, jnp.uint32)
    flip = jnp.where(b >= jnp.uint32(0x80000000),
                     jnp.uint32(0xFFFFFFFF), jnp.uint32(0x80000000))
    return b ^ flip


def _fill(ref, nelem, value, dtype):
    def body(i, _):
        ref[pl.ds(i * 16, 16)] = jnp.full((16,), value, dtype)
        return 0
    lax.fori_loop(0, nelem // 16, body, 0)


def _byte_lookup(words_ref, cell):
    w = plsc.load_gather(words_ref, [cell >> 2])
    return (w >> ((cell & 3) * 8)) & 255


def _p1_body(x_hbm, g_hbm, out_hbm, xv, gv, idxv, onesv, spm):
    c = lax.axis_index("c")
    s = lax.axis_index("s")
    wid = s * NC + c
    dump = H1 + s * 8
    _fill(idxv, CH, 0, jnp.int32)
    pltpu.sync_copy(idxv, spm.at[pl.ds(s * H1S, CH)])
    pltpu.sync_copy(idxv.at[pl.ds(0, H1S - CH)],
                    spm.at[pl.ds(s * H1S + CH, H1S - CH)])
    _fill(onesv, CH, 1, jnp.int32)
    plsc.subcore_barrier()

    def chunk(j, _):
        base = wid * PERW + j * CH
        pltpu.sync_copy(x_hbm.at[pl.ds(base, CH)], xv)
        pltpu.sync_copy(g_hbm.at[pl.ds(base, CH)], gv)

        @plsc.parallel_loop(0, CH // 16, unroll=4)
        def inner(i):
            xk = xv[pl.ds(i * 16, 16)]
            gk = gv[pl.ds(i * 16, 16)]
            key = _key_u32(xk)
            cell = gk * 65536 + (key >> jnp.uint32(16)).astype(jnp.int32)
            cell = jnp.where(xk == 0.0, dump, cell)
            idxv[pl.ds(i * 16, 16)] = cell
        pltpu.sync_copy(onesv, spm.at[idxv], add=True)
        return 0
    lax.fori_loop(0, NCHUNK, chunk, 0)
    plsc.subcore_barrier()
    for off, sz in ((0, 8192), (8192, H1S - 8192)):
        pltpu.sync_copy(spm.at[pl.ds(s * H1S + off, sz)], idxv.at[pl.ds(0, sz)])
        pltpu.sync_copy(idxv.at[pl.ds(0, sz)],
                        out_hbm.at[pl.ds(c * H1P + s * H1S + off, sz)])


def _p2_body(x_hbm, g_hbm, m16_hbm, out_hbm, xv, gv, idxv, onesv, m16v, spm):
    c = lax.axis_index("c")
    s = lax.axis_index("s")
    wid = s * NC + c
    dump = H2 + s * 8
    pltpu.sync_copy(m16_hbm, m16v)
    _fill(idxv, CH, 0, jnp.int32)
    pltpu.sync_copy(idxv.at[pl.ds(0, H2S)], spm.at[pl.ds(s * H2S, H2S)])
    _fill(onesv, CH, 1, jnp.int32)
    plsc.subcore_barrier()

    def chunk(j, _):
        base = wid * PERW + j * CH
        pltpu.sync_copy(x_hbm.at[pl.ds(base, CH)], xv)
        pltpu.sync_copy(g_hbm.at[pl.ds(base, CH)], gv)

        @plsc.parallel_loop(0, CH // 16, unroll=4)
        def inner(i):
            xk = xv[pl.ds(i * 16, 16)]
            gk = gv[pl.ds(i * 16, 16)]
            key = _key_u32(xk)
            c16 = gk * 65536 + (key >> jnp.uint32(16)).astype(jnp.int32)
            v = _byte_lookup(m16v, c16)
            valid = (xk != 0.0) & (v > 0)
            gslot = gk * 128 + v - 1
            cell = gslot * 256 + ((key >> jnp.uint32(8)).astype(jnp.int32) & 255)
            idxv[pl.ds(i * 16, 16)] = jnp.where(valid, cell, dump)
        pltpu.sync_copy(onesv, spm.at[idxv], add=True)
        return 0
    lax.fori_loop(0, NCHUNK, chunk, 0)
    plsc.subcore_barrier()
    pltpu.sync_copy(spm.at[pl.ds(s * H2S, H2S)], idxv.at[pl.ds(0, H2S)])
    pltpu.sync_copy(idxv.at[pl.ds(0, H2S)],
                    out_hbm.at[pl.ds(c * H2P + s * H2S, H2S)])


def _p3_body(x_hbm, g_hbm, m16_hbm, m24_hbm, out_hbm,
             xv, gv, idxv, onesv, m16v, m24v, spm):
    c = lax.axis_index("c")
    s = lax.axis_index("s")
    wid = s * NC + c
    dump = H2 + s * 8
    pltpu.sync_copy(m16_hbm, m16v)
    pltpu.sync_copy(m24_hbm, m24v)
    _fill(idxv, CH3, 0, jnp.int32)
    for q in range(5):
        pltpu.sync_copy(idxv.at[pl.ds(0, H2S // 5)],
                        spm.at[pl.ds(s * H2S + q * (H2S // 5), H2S // 5)])
    _fill(onesv, CH3, 1, jnp.int32)
    plsc.subcore_barrier()

    def cells_for(i):
        xk = xv[pl.ds(i * 16, 16)]
        gk = gv[pl.ds(i * 16, 16)]
        key = _key_u32(xk)
        c16 = gk * 65536 + (key >> jnp.uint32(16)).astype(jnp.int32)
        v = _byte_lookup(m16v, c16)
        valid = (xk != 0.0) & (v > 0)
        gslot = gk * 128 + v - 1
        c24 = gslot * 256 + ((key >> jnp.uint32(8)).astype(jnp.int32) & 255)
        v3 = _byte_lookup(m24v, jnp.where(valid, c24, 0))
        valid = valid & (v3 > 0)
        gslot3 = gk * 128 + v3 - 1
        cell = gslot3 * 256 + (key.astype(jnp.int32) & 255)
        idxv[pl.ds(i * 16, 16)] = jnp.where(valid, cell, dump)

    def chunk(j, _):
        base = wid * PERW + j * CH3
        pltpu.sync_copy(x_hbm.at[pl.ds(base, CH3)], xv)
        pltpu.sync_copy(g_hbm.at[pl.ds(base, CH3)], gv)

        @plsc.parallel_loop(0, CH3 // 16, unroll=4)
        def inner(i):
            cells_for(i)
        pltpu.sync_copy(onesv, spm.at[idxv], add=True)
        return 0
    lax.fori_loop(0, NCHUNK3, chunk, 0)

    @plsc.parallel_loop(0, CH3 // 16, unroll=4)
    def pad(i):
        idxv[pl.ds(i * 16, 16)] = jnp.full((16,), 0, jnp.int32) + dump
    tbase = wid * PERW + NCHUNK3 * CH3
    pltpu.sync_copy(x_hbm.at[pl.ds(tbase, CH3T)], xv.at[pl.ds(0, CH3T)])
    pltpu.sync_copy(g_hbm.at[pl.ds(tbase, CH3T)], gv.at[pl.ds(0, CH3T)])

    @plsc.parallel_loop(0, CH3T // 16, unroll=1)
    def tail(i):
        cells_for(i)
    pltpu.sync_copy(onesv, spm.at[idxv], add=True)
    plsc.subcore_barrier()
    for q in range(5):
        pltpu.sync_copy(spm.at[pl.ds(s * H2S + q * (H2S // 5), H2S // 5)],
                        idxv.at[pl.ds(0, H2S // 5)])
        pltpu.sync_copy(idxv.at[pl.ds(0, H2S // 5)],
                        out_hbm.at[pl.ds(c * H2P + s * H2S + q * (H2S // 5),
                                         H2S // 5)])


NR = 208


def _p4_body(x_hbm, g_hbm, l1_hbm, l2_hbm, l3_hbm, out_hbm,
             xv, gv, outv, l1v, l2v, l3v):
    c = lax.axis_index("c")
    s = lax.axis_index("s")
    wid = s * NC + c
    pltpu.sync_copy(l1_hbm, l1v)
    pltpu.sync_copy(l2_hbm, l2v)
    pltpu.sync_copy(l3_hbm, l3v)

    def chunk(j, _):
        base = wid * PERW + j * CH
        pltpu.sync_copy(x_hbm.at[pl.ds(base, CH)], xv)
        pltpu.sync_copy(g_hbm.at[pl.ds(base, CH)], gv)

        @plsc.parallel_loop(0, CH // 16, unroll=4)
        def inner(i):
            xk = xv[pl.ds(i * 16, 16)]
            gk = gv[pl.ds(i * 16, 16)]
            key = _key_u32(xk)
            c16 = gk * 65536 + (key >> jnp.uint32(16)).astype(jnp.int32)
            v1 = _byte_lookup(l1v, c16)
            mid = (key >> jnp.uint32(8)).astype(jnp.int32) & 255
            t2 = jnp.where(v1 > 50, (v1 - 51) * 256 + mid, 0)
            v2 = _byte_lookup(l2v, t2)
            low = key.astype(jnp.int32) & 255
            t3 = jnp.where(v2 > 50, (v2 - 51) * 256 + low, 0)
            v3 = _byte_lookup(l3v, t3)
            d = jnp.where(v1 <= 50, v1, jnp.where(v2 <= 50, v2, v3))
            outv[pl.ds(i * 16, 16)] = jnp.where(xk != 0.0, d, 0)
        pltpu.sync_copy(outv, out_hbm.at[pl.ds(base, CH)])
        return 0
    lax.fori_loop(0, NCHUNK, chunk, 0)


@functools.cache
def _kernels():
    i32 = jnp.int32
    cp = pltpu.CompilerParams(needs_layout_passes=False)
    p1 = pl.kernel(
        _p1_body,
        out_type=jax.ShapeDtypeStruct((NC * H1P,), i32),
        mesh=_mesh(),
        compiler_params=cp,
        scratch_types=[pltpu.VMEM((CH,), jnp.float32), pltpu.VMEM((CH,), i32),
                       pltpu.VMEM((CH,), i32), pltpu.VMEM((CH,), i32),
                       pltpu.VMEM_SHARED((H1P,), i32)])
    p2 = pl.kernel(
        _p2_body,
        out_type=jax.ShapeDtypeStruct((NC * H2P,), i32),
        mesh=_mesh(),
        compiler_params=cp,
        scratch_types=[pltpu.VMEM((CH,), jnp.float32), pltpu.VMEM((CH,), i32),
                       pltpu.VMEM((CH,), i32), pltpu.VMEM((CH,), i32),
                       pltpu.VMEM((H1 // 4,), i32),
                       pltpu.VMEM_SHARED((H2P,), i32)])
    p3 = pl.kernel(
        _p3_body,
        out_type=jax.ShapeDtypeStruct((NC * H2P,), i32),
        mesh=_mesh(),
        compiler_params=cp,
        scratch_types=[pltpu.VMEM((CH3,), jnp.float32), pltpu.VMEM((CH3,), i32),
                       pltpu.VMEM((CH3,), i32), pltpu.VMEM((CH3,), i32),
                       pltpu.VMEM((H1 // 4,), i32), pltpu.VMEM((H2 // 4,), i32),
                       pltpu.VMEM_SHARED((H2P,), i32)])
    p4 = pl.kernel(
        _p4_body,
        out_type=jax.ShapeDtypeStruct((N,), i32),
        mesh=_mesh(),
        compiler_params=cp,
        scratch_types=[pltpu.VMEM((CH,), jnp.float32), pltpu.VMEM((CH,), i32),
                       pltpu.VMEM((CH,), i32), pltpu.VMEM((H1 // 4,), i32),
                       pltpu.VMEM((NR * 64,), i32), pltpu.VMEM((NR * 64,), i32)])
    return p1, p2, p3, p4


def _unique_pad128(vals):
    sv = jnp.sort(vals, axis=1)
    first = jnp.concatenate(
        [jnp.ones((NG, 1), bool), sv[:, 1:] > sv[:, :-1]], axis=1)
    us = jnp.sort(jnp.where(first, sv, BIG), axis=1)
    cells = jnp.concatenate(
        [us, jnp.full((NG, 128 - 2 * NQ), BIG, jnp.int32)], axis=1)
    slot = jax.vmap(jnp.searchsorted)(cells, vals).astype(jnp.int32)
    return cells, slot


def _pack_bytes(b):
    m = b.reshape(-1, 4)
    return m[:, 0] | (m[:, 1] << 8) | (m[:, 2] << 16) | (m[:, 3] << 24)


def _ss_l(a, v):
    return jnp.searchsorted(a, v, side="left").astype(jnp.int32)


def _ss_r(a, v):
    return jnp.searchsorted(a, v, side="right").astype(jnp.int32)


def _digit_tables(bins):
    eb = lax.bitcast_convert_type(bins, jnp.uint32)
    ekeys = jnp.sort(jnp.where(eb >= jnp.uint32(0x80000000),
                               jnp.uint32(0xFFFFFFFF),
                               jnp.uint32(0x80000000)) ^ eb, axis=1)

    b16 = jnp.arange(65536, dtype=jnp.uint32) << jnp.uint32(16)
    base16 = jax.vmap(lambda e: _ss_l(e, b16))(ekeys)
    cnt16 = jax.vmap(lambda e: _ss_r(e, b16 + jnp.uint32(0xFFFF)))(ekeys) - base16
    amb1 = (cnt16 > 0).reshape(-1)
    rs1 = jnp.cumsum(amb1.astype(jnp.int32)) - 1
    l1 = jnp.where(amb1, 51 + jnp.clip(rs1, 0, NR - 8), base16.reshape(-1))

    flat = jnp.arange(NG * 65536, dtype=jnp.int32)
    tgt = jnp.where(amb1, jnp.clip(rs1, 0, NR - 8), NR - 1)
    gb = jnp.zeros((NR,), jnp.int32).at[tgt].set(flat)
    g1, b1b = gb // 65536, gb % 65536

    mids = jnp.arange(256, dtype=jnp.uint32) << jnp.uint32(8)
    qk2 = (b1b.astype(jnp.uint32) << jnp.uint32(16))[:, None] + mids[None, :]
    ek2 = ekeys[g1]
    base24 = jax.vmap(_ss_l)(ek2, qk2)
    cnt24 = jax.vmap(_ss_r)(ek2, qk2 + jnp.uint32(0xFF)) - base24
    amb2 = (cnt24 > 0).reshape(-1)
    rs2 = jnp.cumsum(amb2.astype(jnp.int32)) - 1
    l2 = jnp.where(amb2, 51 + jnp.clip(rs2, 0, NR - 8), base24.reshape(-1))

    tgt2 = jnp.where(amb2, jnp.clip(rs2, 0, NR - 8), NR - 1)
    qk3 = jnp.zeros((NR,), jnp.uint32).at[tgt2].set(qk2.reshape(-1))
    g2 = jnp.zeros((NR,), jnp.int32).at[tgt2].set(
        jnp.repeat(g1, 256, total_repeat_length=NR * 256))
    lows = jnp.arange(256, dtype=jnp.uint32)
    fk = qk3[:, None] + lows[None, :]
    l3 = jax.vmap(_ss_r)(ekeys[g2], fk).reshape(-1)

    return _pack_bytes(l1), _pack_bytes(l2), _pack_bytes(l3)


def _pack_map(total, idx, valid):
    vals = jnp.where(valid, jnp.arange(1, 129, dtype=jnp.int32)[None, :], 0)
    safe = jnp.where(valid, idx, total).reshape(-1)
    m = jnp.zeros((total + 8,), jnp.int32).at[safe].set(vals.reshape(-1))
    m = m[:total].reshape(-1, 4)
    return m[:, 0] | (m[:, 1] << 8) | (m[:, 2] << 16) | (m[:, 3] << 24)


def _rank_step(csum_rows, ranks):
    nb = csum_rows.shape[1]
    b = jax.vmap(lambda a, v: jnp.searchsorted(a, v, side="right"))(
        csum_rows, ranks).astype(jnp.int32)
    b = jnp.clip(b, 0, nb - 1)
    prev = jnp.take_along_axis(csum_rows, jnp.clip(b - 1, 0, nb - 1)[..., None],
                               axis=1)[..., 0] if ranks.ndim == 1 else None
    if prev is None:
        prev = jnp.take_along_axis(csum_rows, jnp.clip(b - 1, 0, nb - 1), axis=1)
    prev = jnp.where(b > 0, prev, 0)
    return b, ranks - prev


def kernel(x, group):
    p1, p2, p3, p4 = _kernels()

    h1 = p1(x, group)
    h1 = h1.reshape(NC, H1P)[:, :H1].sum(0).reshape(NG, 65536)
    csum1 = jnp.cumsum(h1, axis=1)
    n_g = csum1[:, -1]

    qs = jnp.linspace(0.0, 1.0, NBINS - 1)
    pos = qs[None, :] * jnp.maximum(n_g - 1, 0).astype(jnp.float32)[:, None]
    lo = jnp.clip(jnp.floor(pos).astype(jnp.int32), 0, N - 1)
    hi = jnp.clip(jnp.ceil(pos).astype(jnp.int32), 0, N - 1)
    frac = pos - jnp.floor(pos)
    ranks = jnp.stack([lo, hi], axis=-1).reshape(NG, 2 * NQ)

    b1, r1 = _rank_step(csum1, ranks)
    cells2, slot2 = _unique_pad128(b1)
    gslot = jnp.arange(NG, dtype=jnp.int32)[:, None] * 128 + slot2
    m16 = _pack_map(H1, jnp.arange(NG, dtype=jnp.int32)[:, None] * 65536 + cells2,
                    cells2 < BIG)

    h2 = p2(x, group, m16)
    h2 = h2.reshape(NC, H2P)[:, :H2].sum(0).reshape(S2, 256)
    csum2 = jnp.cumsum(h2, axis=1)
    rows2 = csum2[gslot.reshape(-1)]
    b2, r2 = _rank_step(rows2, r1.reshape(-1)[:, None])
    b2, r2 = b2[:, 0], r2[:, 0]

    cell24 = (gslot.reshape(-1) * 256 + b2).reshape(NG, 2 * NQ)
    cells3, slot3 = _unique_pad128(cell24)
    gslot3 = jnp.arange(NG, dtype=jnp.int32)[:, None] * 128 + slot3
    m24 = _pack_map(H2, cells3, cells3 < BIG)

    h3 = p3(x, group, m16, m24)
    h3 = h3.reshape(NC, H2P)[:, :H2].sum(0).reshape(S2, 256)
    csum3 = jnp.cumsum(h3, axis=1)
    rows3 = csum3[gslot3.reshape(-1)]
    b3, _ = _rank_step(rows3, r2[:, None])
    b3 = b3[:, 0]

    keyfull = ((b1.reshape(-1).astype(jnp.uint32) << jnp.uint32(16))
               | (b2.astype(jnp.uint32) << jnp.uint32(8))
               | b3.astype(jnp.uint32))
    fb = jnp.where(keyfull >= jnp.uint32(0x80000000),
                   keyfull ^ jnp.uint32(0x80000000), ~keyfull)
    svals = lax.bitcast_convert_type(fb, jnp.float32).reshape(NG, NQ, 2)
    s_lo, s_hi = svals[:, :, 0], svals[:, :, 1]
    bins = s_lo * (1.0 - frac) + s_hi * frac
    l1, l2, l3 = _digit_tables(bins)
    return p4(x, group, l1, l2, l3)


# speedup vs baseline: 1.0356x; 1.0356x over previous
import functools

import jax
import jax.numpy as jnp
from jax import lax
from jax.experimental import pallas as pl
from jax.experimental.pallas import tpu as pltpu
from jax.experimental.pallas import tpu_sc as plsc

N = 8_000_000
NG = 4
NBINS = 51
NQ = NBINS - 1
NC = 2
NS = 16
NW = NC * NS
PERW = N // NW
CH = 10_000
NCHUNK = PERW // CH
CH3 = 5_952
NCHUNK3 = PERW // CH3
CH3T = PERW - NCHUNK3 * CH3

H1 = NG * 65536
H1P = H1 + 128
H1S = H1P // NS
S2 = NG * 128
H2 = S2 * 256
H2P = H2 + 128
H2S = H2P // NS
BIG = 2**31 - 1

@functools.cache
def _mesh():
    return plsc.VectorSubcoreMesh(
        core_axis_name="c", subcore_axis_name="s",
        num_cores=NC, num_subcores=NS)


def _key_u32(xk):
    b = lax.bitcast_convert_type(xk, jnp.uint32)
    flip = jnp.where(b >= jnp.uint32(0x80000000),
                     jnp.uint32(0xFFFFFFFF), jnp.uint32(0x80000000))
    return b ^ flip


def _fill(ref, nelem, value, dtype):
    def body(i, _):
        ref[pl.ds(i * 16, 16)] = jnp.full((16,), value, dtype)
        return 0
    lax.fori_loop(0, nelem // 16, body, 0)


def _byte_lookup(words_ref, cell):
    w = plsc.load_gather(words_ref, [cell >> 2])
    return (w >> ((cell & 3) * 8)) & 255


def _p1_body(x_hbm, g_hbm, out_hbm, xv, gv, idxv, onesv, spm):
    c = lax.axis_index("c")
    s = lax.axis_index("s")
    wid = s * NC + c
    dump = H1 + s * 8
    _fill(idxv, CH, 0, jnp.int32)
    pltpu.sync_copy(idxv, spm.at[pl.ds(s * H1S, CH)])
    pltpu.sync_copy(idxv.at[pl.ds(0, H1S - CH)],
                    spm.at[pl.ds(s * H1S + CH, H1S - CH)])
    _fill(onesv, CH, 1, jnp.int32)
    plsc.subcore_barrier()

    def chunk(j, _):
        base = wid * PERW + j * CH
        pltpu.sync_copy(x_hbm.at[pl.ds(base, CH)], xv)
        pltpu.sync_copy(g_hbm.at[pl.ds(base, CH)], gv)

        @plsc.parallel_loop(0, CH // 16, unroll=4)
        def inner(i):
            xk = xv[pl.ds(i * 16, 16)]
            gk = gv[pl.ds(i * 16, 16)]
            key = _key_u32(xk)
            cell = gk * 65536 + (key >> jnp.uint32(16)).astype(jnp.int32)
            cell = jnp.where(xk == 0.0, dump, cell)
            idxv[pl.ds(i * 16, 16)] = cell
        pltpu.sync_copy(onesv, spm.at[idxv], add=True)
        return 0
    lax.fori_loop(0, NCHUNK, chunk, 0)
    plsc.subcore_barrier()
    for off, sz in ((0, 8192), (8192, H1S - 8192)):
        pltpu.sync_copy(spm.at[pl.ds(s * H1S + off, sz)], idxv.at[pl.ds(0, sz)])
        pltpu.sync_copy(idxv.at[pl.ds(0, sz)],
                        out_hbm.at[pl.ds(c * H1P + s * H1S + off, sz)])


def _p2_body(x_hbm, g_hbm, m16_hbm, out_hbm, xv, gv, idxv, onesv, m16v, spm):
    c = lax.axis_index("c")
    s = lax.axis_index("s")
    wid = s * NC + c
    dump = H2 + s * 8
    pltpu.sync_copy(m16_hbm, m16v)
    _fill(idxv, CH, 0, jnp.int32)
    pltpu.sync_copy(idxv.at[pl.ds(0, H2S)], spm.at[pl.ds(s * H2S, H2S)])
    _fill(onesv, CH, 1, jnp.int32)
    plsc.subcore_barrier()

    def chunk(j, _):
        base = wid * PERW + j * CH
        pltpu.sync_copy(x_hbm.at[pl.ds(base, CH)], xv)
        pltpu.sync_copy(g_hbm.at[pl.ds(base, CH)], gv)

        @plsc.parallel_loop(0, CH // 16, unroll=4)
        def inner(i):
            xk = xv[pl.ds(i * 16, 16)]
            gk = gv[pl.ds(i * 16, 16)]
            key = _key_u32(xk)
            c16 = gk * 65536 + (key >> jnp.uint32(16)).astype(jnp.int32)
            v = _byte_lookup(m16v, c16)
            valid = (xk != 0.0) & (v > 0)
            gslot = gk * 128 + v - 1
            cell = gslot * 256 + ((key >> jnp.uint32(8)).astype(jnp.int32) & 255)
            idxv[pl.ds(i * 16, 16)] = jnp.where(valid, cell, dump)
        pltpu.sync_copy(onesv, spm.at[idxv], add=True)
        return 0
    lax.fori_loop(0, NCHUNK, chunk, 0)
    plsc.subcore_barrier()
    pltpu.sync_copy(spm.at[pl.ds(s * H2S, H2S)], idxv.at[pl.ds(0, H2S)])
    pltpu.sync_copy(idxv.at[pl.ds(0, H2S)],
                    out_hbm.at[pl.ds(c * H2P + s * H2S, H2S)])


def _p3_body(x_hbm, g_hbm, m16_hbm, m24_hbm, out_hbm,
             xv, gv, idxv, onesv, m16v, m24v, spm):
    c = lax.axis_index("c")
    s = lax.axis_index("s")
    wid = s * NC + c
    dump = H2 + s * 8
    pltpu.sync_copy(m16_hbm, m16v)
    pltpu.sync_copy(m24_hbm, m24v)
    _fill(idxv, CH3, 0, jnp.int32)
    for q in range(5):
        pltpu.sync_copy(idxv.at[pl.ds(0, H2S // 5)],
                        spm.at[pl.ds(s * H2S + q * (H2S // 5), H2S // 5)])
    _fill(onesv, CH3, 1, jnp.int32)
    plsc.subcore_barrier()

    def cells_for(i):
        xk = xv[pl.ds(i * 16, 16)]
        gk = gv[pl.ds(i * 16, 16)]
        key = _key_u32(xk)
        c16 = gk * 65536 + (key >> jnp.uint32(16)).astype(jnp.int32)
        v = _byte_lookup(m16v, c16)
        valid = (xk != 0.0) & (v > 0)
        gslot = gk * 128 + v - 1
        c24 = gslot * 256 + ((key >> jnp.uint32(8)).astype(jnp.int32) & 255)
        v3 = _byte_lookup(m24v, jnp.where(valid, c24, 0))
        valid = valid & (v3 > 0)
        gslot3 = gk * 128 + v3 - 1
        cell = gslot3 * 256 + (key.astype(jnp.int32) & 255)
        idxv[pl.ds(i * 16, 16)] = jnp.where(valid, cell, dump)

    def chunk(j, _):
        base = wid * PERW + j * CH3
        pltpu.sync_copy(x_hbm.at[pl.ds(base, CH3)], xv)
        pltpu.sync_copy(g_hbm.at[pl.ds(base, CH3)], gv)

        @plsc.parallel_loop(0, CH3 // 16, unroll=4)
        def inner(i):
            cells_for(i)
        pltpu.sync_copy(onesv, spm.at[idxv], add=True)
        return 0
    lax.fori_loop(0, NCHUNK3, chunk, 0)

    @plsc.parallel_loop(0, CH3 // 16, unroll=4)
    def pad(i):
        idxv[pl.ds(i * 16, 16)] = jnp.full((16,), 0, jnp.int32) + dump
    tbase = wid * PERW + NCHUNK3 * CH3
    pltpu.sync_copy(x_hbm.at[pl.ds(tbase, CH3T)], xv.at[pl.ds(0, CH3T)])
    pltpu.sync_copy(g_hbm.at[pl.ds(tbase, CH3T)], gv.at[pl.ds(0, CH3T)])

    @plsc.parallel_loop(0, CH3T // 16, unroll=1)
    def tail(i):
        cells_for(i)
    pltpu.sync_copy(onesv, spm.at[idxv], add=True)
    plsc.subcore_barrier()
    for q in range(5):
        pltpu.sync_copy(spm.at[pl.ds(s * H2S + q * (H2S // 5), H2S // 5)],
                        idxv.at[pl.ds(0, H2S // 5)])
        pltpu.sync_copy(idxv.at[pl.ds(0, H2S // 5)],
                        out_hbm.at[pl.ds(c * H2P + s * H2S + q * (H2S // 5),
                                         H2S // 5)])


NR = 208


def _p4_body(x_hbm, g_hbm, l1_hbm, l2_hbm, l3_hbm, out_hbm,
             xv, gv, outv, l1v, l2v, l3v):
    c = lax.axis_index("c")
    s = lax.axis_index("s")
    wid = s * NC + c
    pltpu.sync_copy(l1_hbm, l1v)
    pltpu.sync_copy(l2_hbm, l2v)
    pltpu.sync_copy(l3_hbm, l3v)

    def chunk(j, _):
        base = wid * PERW + j * CH
        pltpu.sync_copy(x_hbm.at[pl.ds(base, CH)], xv)
        pltpu.sync_copy(g_hbm.at[pl.ds(base, CH)], gv)

        @plsc.parallel_loop(0, CH // 16, unroll=4)
        def inner(i):
            xk = xv[pl.ds(i * 16, 16)]
            gk = gv[pl.ds(i * 16, 16)]
            key = _key_u32(xk)
            c16 = gk * 65536 + (key >> jnp.uint32(16)).astype(jnp.int32)
            v1 = _byte_lookup(l1v, c16)
            mid = (key >> jnp.uint32(8)).astype(jnp.int32) & 255
            t2 = jnp.where(v1 > 50, (v1 - 51) * 256 + mid, 0)
            v2 = _byte_lookup(l2v, t2)
            low = key.astype(jnp.int32) & 255
            t3 = jnp.where(v2 > 50, (v2 - 51) * 256 + low, 0)
            v3 = _byte_lookup(l3v, t3)
            d = jnp.where(v1 <= 50, v1, jnp.where(v2 <= 50, v2, v3))
            outv[pl.ds(i * 16, 16)] = jnp.where(xk != 0.0, d, 0)
        pltpu.sync_copy(outv, out_hbm.at[pl.ds(base, CH)])
        return 0
    lax.fori_loop(0, NCHUNK, chunk, 0)


@functools.cache
def _kernels():
    i32 = jnp.int32
    cp = pltpu.CompilerParams(needs_layout_passes=False)
    p1 = pl.kernel(
        _p1_body,
        out_type=jax.ShapeDtypeStruct((NC * H1P,), i32),
        mesh=_mesh(),
        compiler_params=cp,
        scratch_types=[pltpu.VMEM((CH,), jnp.float32), pltpu.VMEM((CH,), i32),
                       pltpu.VMEM((CH,), i32), pltpu.VMEM((CH,), i32),
                       pltpu.VMEM_SHARED((H1P,), i32)])
    p2 = pl.kernel(
        _p2_body,
        out_type=jax.ShapeDtypeStruct((NC * H2P,), i32),
        mesh=_mesh(),
        compiler_params=cp,
        scratch_types=[pltpu.VMEM((CH,), jnp.float32), pltpu.VMEM((CH,), i32),
                       pltpu.VMEM((CH,), i32), pltpu.VMEM((CH,), i32),
                       pltpu.VMEM((H1 // 4,), i32),
                       pltpu.VMEM_SHARED((H2P,), i32)])
    p3 = pl.kernel(
        _p3_body,
        out_type=jax.ShapeDtypeStruct((NC * H2P,), i32),
        mesh=_mesh(),
        compiler_params=cp,
        scratch_types=[pltpu.VMEM((CH3,), jnp.float32), pltpu.VMEM((CH3,), i32),
                       pltpu.VMEM((CH3,), i32), pltpu.VMEM((CH3,), i32),
                       pltpu.VMEM((H1 // 4,), i32), pltpu.VMEM((H2 // 4,), i32),
                       pltpu.VMEM_SHARED((H2P,), i32)])
    p4 = pl.kernel(
        _p4_body,
        out_type=jax.ShapeDtypeStruct((N,), i32),
        mesh=_mesh(),
        compiler_params=cp,
        scratch_types=[pltpu.VMEM((CH,), jnp.float32), pltpu.VMEM((CH,), i32),
                       pltpu.VMEM((CH,), i32), pltpu.VMEM((H1 // 4,), i32),
                       pltpu.VMEM((NR * 64,), i32), pltpu.VMEM((NR * 64,), i32)])
    return p1, p2, p3, p4


def _unique_pad128(vals):
    sv = jnp.sort(vals, axis=1)
    first = jnp.concatenate(
        [jnp.ones((NG, 1), bool), sv[:, 1:] > sv[:, :-1]], axis=1)
    us = jnp.sort(jnp.where(first, sv, BIG), axis=1)
    cells = jnp.concatenate(
        [us, jnp.full((NG, 128 - 2 * NQ), BIG, jnp.int32)], axis=1)
    slot = jax.vmap(jnp.searchsorted)(cells, vals).astype(jnp.int32)
    return cells, slot


def _pack_bytes(b):
    m = b.reshape(-1, 4)
    return m[:, 0] | (m[:, 1] << 8) | (m[:, 2] << 16) | (m[:, 3] << 24)


def _ss_l(a, v):
    return jnp.searchsorted(a, v, side="left").astype(jnp.int32)


def _ss_r(a, v):
    return jnp.searchsorted(a, v, side="right").astype(jnp.int32)


def _digit_tables(bins):
    eb = lax.bitcast_convert_type(bins, jnp.uint32)
    ekeys = jnp.sort(jnp.where(eb >= jnp.uint32(0x80000000),
                               jnp.uint32(0xFFFFFFFF),
                               jnp.uint32(0x80000000)) ^ eb, axis=1)

    b16 = jnp.arange(65536, dtype=jnp.uint32) << jnp.uint32(16)
    base16 = jax.vmap(lambda e: _ss_l(e, b16))(ekeys)
    cnt16 = jax.vmap(lambda e: _ss_r(e, b16 + jnp.uint32(0xFFFF)))(ekeys) - base16
    amb1 = (cnt16 > 0).reshape(-1)
    rs1 = jnp.cumsum(amb1.astype(jnp.int32)) - 1
    l1 = jnp.where(amb1, 51 + jnp.clip(rs1, 0, NR - 8), base16.reshape(-1))

    flat = jnp.arange(NG * 65536, dtype=jnp.int32)
    gb = -lax.top_k(-jnp.where(amb1, flat, BIG - NR), NR)[0]
    gb = jnp.clip(gb, 0, NG * 65536 - 1)
    g1, b1b = gb // 65536, gb % 65536

    mids = jnp.arange(256, dtype=jnp.uint32) << jnp.uint32(8)
    qk2 = (b1b.astype(jnp.uint32) << jnp.uint32(16))[:, None] + mids[None, :]
    ek2 = ekeys[g1]
    base24 = jax.vmap(_ss_l)(ek2, qk2)
    cnt24 = jax.vmap(_ss_r)(ek2, qk2 + jnp.uint32(0xFF)) - base24
    amb2 = (cnt24 > 0).reshape(-1)
    rs2 = jnp.cumsum(amb2.astype(jnp.int32)) - 1
    l2 = jnp.where(amb2, 51 + jnp.clip(rs2, 0, NR - 8), base24.reshape(-1))

    flat2 = jnp.arange(NR * 256, dtype=jnp.int32)
    sel2 = -lax.top_k(-jnp.where(amb2, flat2, BIG - NR), NR)[0]
    sel2 = jnp.clip(sel2, 0, NR * 256 - 1)
    qk3 = qk2.reshape(-1)[sel2]
    g2 = g1[sel2 // 256]
    lows = jnp.arange(256, dtype=jnp.uint32)
    fk = qk3[:, None] + lows[None, :]
    l3 = jax.vmap(_ss_r)(ekeys[g2], fk).reshape(-1)

    return _pack_bytes(l1), _pack_bytes(l2), _pack_bytes(l3)


def _pack_map(total, idx, valid):
    vals = jnp.where(valid, jnp.arange(1, 129, dtype=jnp.int32)[None, :], 0)
    safe = jnp.where(valid, idx, total).reshape(-1)
    m = jnp.zeros((total + 8,), jnp.int32).at[safe].set(vals.reshape(-1))
    m = m[:total].reshape(-1, 4)
    return m[:, 0] | (m[:, 1] << 8) | (m[:, 2] << 16) | (m[:, 3] << 24)


def _rank_step(csum_rows, ranks):
    nb = csum_rows.shape[1]
    b = jax.vmap(lambda a, v: jnp.searchsorted(a, v, side="right"))(
        csum_rows, ranks).astype(jnp.int32)
    b = jnp.clip(b, 0, nb - 1)
    prev = jnp.take_along_axis(csum_rows, jnp.clip(b - 1, 0, nb - 1)[..., None],
                               axis=1)[..., 0] if ranks.ndim == 1 else None
    if prev is None:
        prev = jnp.take_along_axis(csum_rows, jnp.clip(b - 1, 0, nb - 1), axis=1)
    prev = jnp.where(b > 0, prev, 0)
    return b, ranks - prev


def kernel(x, group):
    p1, p2, p3, p4 = _kernels()

    h1 = p1(x, group)
    h1 = h1.reshape(NC, H1P)[:, :H1].sum(0).reshape(NG, 65536)
    csum1 = jnp.cumsum(h1, axis=1)
    n_g = csum1[:, -1]

    qs = jnp.linspace(0.0, 1.0, NBINS - 1)
    pos = qs[None, :] * jnp.maximum(n_g - 1, 0).astype(jnp.float32)[:, None]
    lo = jnp.clip(jnp.floor(pos).astype(jnp.int32), 0, N - 1)
    hi = jnp.clip(jnp.ceil(pos).astype(jnp.int32), 0, N - 1)
    frac = pos - jnp.floor(pos)
    ranks = jnp.stack([lo, hi], axis=-1).reshape(NG, 2 * NQ)

    b1, r1 = _rank_step(csum1, ranks)
    cells2, slot2 = _unique_pad128(b1)
    gslot = jnp.arange(NG, dtype=jnp.int32)[:, None] * 128 + slot2
    m16 = _pack_map(H1, jnp.arange(NG, dtype=jnp.int32)[:, None] * 65536 + cells2,
                    cells2 < BIG)

    h2 = p2(x, group, m16)
    h2 = h2.reshape(NC, H2P)[:, :H2].sum(0).reshape(S2, 256)
    csum2 = jnp.cumsum(h2, axis=1)
    rows2 = csum2[gslot.reshape(-1)]
    b2, r2 = _rank_step(rows2, r1.reshape(-1)[:, None])
    b2, r2 = b2[:, 0], r2[:, 0]

    cell24 = (gslot.reshape(-1) * 256 + b2).reshape(NG, 2 * NQ)
    cells3, slot3 = _unique_pad128(cell24)
    gslot3 = jnp.arange(NG, dtype=jnp.int32)[:, None] * 128 + slot3
    m24 = _pack_map(H2, cells3, cells3 < BIG)

    h3 = p3(x, group, m16, m24)
    h3 = h3.reshape(NC, H2P)[:, :H2].sum(0).reshape(S2, 256)
    csum3 = jnp.cumsum(h3, axis=1)
    rows3 = csum3[gslot3.reshape(-1)]
    b3, _ = _rank_step(rows3, r2[:, None])
    b3 = b3[:, 0]

    keyfull = ((b1.reshape(-1).astype(jnp.uint32) << jnp.uint32(16))
               | (b2.astype(jnp.uint32) << jnp.uint32(8))
               | b3.astype(jnp.uint32))
    fb = jnp.where(keyfull >= jnp.uint32(0x80000000),
                   keyfull ^ jnp.uint32(0x80000000), ~keyfull)
    svals = lax.bitcast_convert_type(fb, jnp.float32).reshape(NG, NQ, 2)
    s_lo, s_hi = svals[:, :, 0], svals[:, :, 1]
    bins = s_lo * (1.0 - frac) + s_hi * frac
    l1, l2, l3 = _digit_tables(bins)
    return p4(x, group, l1, l2, l3)


# speedup vs baseline: 24.6947x; 23.8462x over previous
import functools

import jax
import jax.numpy as jnp
from jax import lax
from jax.experimental import pallas as pl
from jax.experimental.pallas import tpu as pltpu
from jax.experimental.pallas import tpu_sc as plsc

N = 8_000_000
NG = 4
NBINS = 51
NQ = NBINS - 1
NC = 2
NS = 16
NW = NC * NS
PERW = N // NW
CH = 10_000
NCHUNK = PERW // CH
CH3 = 5_952
NCHUNK3 = PERW // CH3
CH3T = PERW - NCHUNK3 * CH3

H1 = NG * 65536
H1P = H1 + 128
H1S = H1P // NS
S2 = NG * 128
H2 = S2 * 256
H2P = H2 + 128
H2S = H2P // NS
BIG = 2**31 - 1

@functools.cache
def _mesh():
    return plsc.VectorSubcoreMesh(
        core_axis_name="c", subcore_axis_name="s",
        num_cores=NC, num_subcores=NS)


def _key_u32(xk):
    b = lax.bitcast_convert_type(xk, jnp.uint32)
    flip = jnp.where(b >= jnp.uint32(0x80000000),
                     jnp.uint32(0xFFFFFFFF), jnp.uint32(0x80000000))
    return b ^ flip


def _fill(ref, nelem, value, dtype):
    def body(i, _):
        ref[pl.ds(i * 16, 16)] = jnp.full((16,), value, dtype)
        return 0
    lax.fori_loop(0, nelem // 16, body, 0)


def _byte_lookup(words_ref, cell):
    w = plsc.load_gather(words_ref, [cell >> 2])
    return (w >> ((cell & 3) * 8)) & 255


def _p1_body(x_hbm, g_hbm, out_hbm, xv, gv, idxv, onesv, spm):
    c = lax.axis_index("c")
    s = lax.axis_index("s")
    wid = s * NC + c
    dump = H1 + s * 8
    _fill(idxv, CH, 0, jnp.int32)
    pltpu.sync_copy(idxv, spm.at[pl.ds(s * H1S, CH)])
    pltpu.sync_copy(idxv.at[pl.ds(0, H1S - CH)],
                    spm.at[pl.ds(s * H1S + CH, H1S - CH)])
    _fill(onesv, CH, 1, jnp.int32)
    plsc.subcore_barrier()

    def chunk(j, _):
        base = wid * PERW + j * CH
        pltpu.sync_copy(x_hbm.at[pl.ds(base, CH)], xv)
        pltpu.sync_copy(g_hbm.at[pl.ds(base, CH)], gv)

        @plsc.parallel_loop(0, CH // 16, unroll=4)
        def inner(i):
            xk = xv[pl.ds(i * 16, 16)]
            gk = gv[pl.ds(i * 16, 16)]
            key = _key_u32(xk)
            cell = gk * 65536 + (key >> jnp.uint32(16)).astype(jnp.int32)
            cell = jnp.where(xk == 0.0, dump, cell)
            idxv[pl.ds(i * 16, 16)] = cell
        pltpu.sync_copy(onesv, spm.at[idxv], add=True)
        return 0
    lax.fori_loop(0, NCHUNK, chunk, 0)
    plsc.subcore_barrier()
    for off, sz in ((0, 8192), (8192, H1S - 8192)):
        pltpu.sync_copy(spm.at[pl.ds(s * H1S + off, sz)], idxv.at[pl.ds(0, sz)])
        pltpu.sync_copy(idxv.at[pl.ds(0, sz)],
                        out_hbm.at[pl.ds(c * H1P + s * H1S + off, sz)])


def _p2_body(x_hbm, g_hbm, m16_hbm, out_hbm, xv, gv, idxv, onesv, m16v, spm):
    c = lax.axis_index("c")
    s = lax.axis_index("s")
    wid = s * NC + c
    dump = H2 + s * 8
    pltpu.sync_copy(m16_hbm, m16v)
    _fill(idxv, CH, 0, jnp.int32)
    pltpu.sync_copy(idxv.at[pl.ds(0, H2S)], spm.at[pl.ds(s * H2S, H2S)])
    _fill(onesv, CH, 1, jnp.int32)
    plsc.subcore_barrier()

    def chunk(j, _):
        base = wid * PERW + j * CH
        pltpu.sync_copy(x_hbm.at[pl.ds(base, CH)], xv)
        pltpu.sync_copy(g_hbm.at[pl.ds(base, CH)], gv)

        @plsc.parallel_loop(0, CH // 16, unroll=4)
        def inner(i):
            xk = xv[pl.ds(i * 16, 16)]
            gk = gv[pl.ds(i * 16, 16)]
            key = _key_u32(xk)
            c16 = gk * 65536 + (key >> jnp.uint32(16)).astype(jnp.int32)
            v = _byte_lookup(m16v, c16)
            valid = (xk != 0.0) & (v > 0)
            gslot = gk * 128 + v - 1
            cell = gslot * 256 + ((key >> jnp.uint32(8)).astype(jnp.int32) & 255)
            idxv[pl.ds(i * 16, 16)] = jnp.where(valid, cell, dump)
        pltpu.sync_copy(onesv, spm.at[idxv], add=True)
        return 0
    lax.fori_loop(0, NCHUNK, chunk, 0)
    plsc.subcore_barrier()
    pltpu.sync_copy(spm.at[pl.ds(s * H2S, H2S)], idxv.at[pl.ds(0, H2S)])
    pltpu.sync_copy(idxv.at[pl.ds(0, H2S)],
                    out_hbm.at[pl.ds(c * H2P + s * H2S, H2S)])


def _p3_body(x_hbm, g_hbm, m16_hbm, m24_hbm, out_hbm,
             xv, gv, idxv, onesv, m16v, m24v, spm):
    c = lax.axis_index("c")
    s = lax.axis_index("s")
    wid = s * NC + c
    dump = H2 + s * 8
    pltpu.sync_copy(m16_hbm, m16v)
    pltpu.sync_copy(m24_hbm, m24v)
    _fill(idxv, CH3, 0, jnp.int32)
    for q in range(5):
        pltpu.sync_copy(idxv.at[pl.ds(0, H2S // 5)],
                        spm.at[pl.ds(s * H2S + q * (H2S // 5), H2S // 5)])
    _fill(onesv, CH3, 1, jnp.int32)
    plsc.subcore_barrier()

    def cells_for(i):
        xk = xv[pl.ds(i * 16, 16)]
        gk = gv[pl.ds(i * 16, 16)]
        key = _key_u32(xk)
        c16 = gk * 65536 + (key >> jnp.uint32(16)).astype(jnp.int32)
        v = _byte_lookup(m16v, c16)
        valid = (xk != 0.0) & (v > 0)
        gslot = gk * 128 + v - 1
        c24 = gslot * 256 + ((key >> jnp.uint32(8)).astype(jnp.int32) & 255)
        v3 = _byte_lookup(m24v, jnp.where(valid, c24, 0))
        valid = valid & (v3 > 0)
        gslot3 = gk * 128 + v3 - 1
        cell = gslot3 * 256 + (key.astype(jnp.int32) & 255)
        idxv[pl.ds(i * 16, 16)] = jnp.where(valid, cell, dump)

    def chunk(j, _):
        base = wid * PERW + j * CH3
        pltpu.sync_copy(x_hbm.at[pl.ds(base, CH3)], xv)
        pltpu.sync_copy(g_hbm.at[pl.ds(base, CH3)], gv)

        @plsc.parallel_loop(0, CH3 // 16, unroll=4)
        def inner(i):
            cells_for(i)
        pltpu.sync_copy(onesv, spm.at[idxv], add=True)
        return 0
    lax.fori_loop(0, NCHUNK3, chunk, 0)

    @plsc.parallel_loop(0, CH3 // 16, unroll=4)
    def pad(i):
        idxv[pl.ds(i * 16, 16)] = jnp.full((16,), 0, jnp.int32) + dump
    tbase = wid * PERW + NCHUNK3 * CH3
    pltpu.sync_copy(x_hbm.at[pl.ds(tbase, CH3T)], xv.at[pl.ds(0, CH3T)])
    pltpu.sync_copy(g_hbm.at[pl.ds(tbase, CH3T)], gv.at[pl.ds(0, CH3T)])

    @plsc.parallel_loop(0, CH3T // 16, unroll=1)
    def tail(i):
        cells_for(i)
    pltpu.sync_copy(onesv, spm.at[idxv], add=True)
    plsc.subcore_barrier()
    for q in range(5):
        pltpu.sync_copy(spm.at[pl.ds(s * H2S + q * (H2S // 5), H2S // 5)],
                        idxv.at[pl.ds(0, H2S // 5)])
        pltpu.sync_copy(idxv.at[pl.ds(0, H2S // 5)],
                        out_hbm.at[pl.ds(c * H2P + s * H2S + q * (H2S // 5),
                                         H2S // 5)])


NR = 208


def _p4_body(x_hbm, g_hbm, l1_hbm, l2_hbm, l3_hbm, out_hbm,
             xv, gv, outv, l1v, l2v, l3v):
    c = lax.axis_index("c")
    s = lax.axis_index("s")
    wid = s * NC + c
    pltpu.sync_copy(l1_hbm, l1v)
    pltpu.sync_copy(l2_hbm, l2v)
    pltpu.sync_copy(l3_hbm, l3v)

    def chunk(j, _):
        base = wid * PERW + j * CH
        pltpu.sync_copy(x_hbm.at[pl.ds(base, CH)], xv)
        pltpu.sync_copy(g_hbm.at[pl.ds(base, CH)], gv)

        @plsc.parallel_loop(0, CH // 16, unroll=4)
        def inner(i):
            xk = xv[pl.ds(i * 16, 16)]
            gk = gv[pl.ds(i * 16, 16)]
            key = _key_u32(xk)
            c16 = gk * 65536 + (key >> jnp.uint32(16)).astype(jnp.int32)
            v1 = _byte_lookup(l1v, c16)
            mid = (key >> jnp.uint32(8)).astype(jnp.int32) & 255
            t2 = jnp.where(v1 > 50, (v1 - 51) * 256 + mid, 0)
            v2 = _byte_lookup(l2v, t2)
            low = key.astype(jnp.int32) & 255
            t3 = jnp.where(v2 > 50, (v2 - 51) * 256 + low, 0)
            v3 = _byte_lookup(l3v, t3)
            d = jnp.where(v1 <= 50, v1, jnp.where(v2 <= 50, v2, v3))
            outv[pl.ds(i * 16, 16)] = jnp.where(xk != 0.0, d, 0)
        pltpu.sync_copy(outv, out_hbm.at[pl.ds(base, CH)])
        return 0
    lax.fori_loop(0, NCHUNK, chunk, 0)


@functools.cache
def _kernels():
    i32 = jnp.int32
    cp = pltpu.CompilerParams(needs_layout_passes=False)
    p1 = pl.kernel(
        _p1_body,
        out_type=jax.ShapeDtypeStruct((NC * H1P,), i32),
        mesh=_mesh(),
        compiler_params=cp,
        scratch_types=[pltpu.VMEM((CH,), jnp.float32), pltpu.VMEM((CH,), i32),
                       pltpu.VMEM((CH,), i32), pltpu.VMEM((CH,), i32),
                       pltpu.VMEM_SHARED((H1P,), i32)])
    p2 = pl.kernel(
        _p2_body,
        out_type=jax.ShapeDtypeStruct((NC * H2P,), i32),
        mesh=_mesh(),
        compiler_params=cp,
        scratch_types=[pltpu.VMEM((CH,), jnp.float32), pltpu.VMEM((CH,), i32),
                       pltpu.VMEM((CH,), i32), pltpu.VMEM((CH,), i32),
                       pltpu.VMEM((H1 // 4,), i32),
                       pltpu.VMEM_SHARED((H2P,), i32)])
    p3 = pl.kernel(
        _p3_body,
        out_type=jax.ShapeDtypeStruct((NC * H2P,), i32),
        mesh=_mesh(),
        compiler_params=cp,
        scratch_types=[pltpu.VMEM((CH3,), jnp.float32), pltpu.VMEM((CH3,), i32),
                       pltpu.VMEM((CH3,), i32), pltpu.VMEM((CH3,), i32),
                       pltpu.VMEM((H1 // 4,), i32), pltpu.VMEM((H2 // 4,), i32),
                       pltpu.VMEM_SHARED((H2P,), i32)])
    p4 = pl.kernel(
        _p4_body,
        out_type=jax.ShapeDtypeStruct((N,), i32),
        mesh=_mesh(),
        compiler_params=cp,
        scratch_types=[pltpu.VMEM((CH,), jnp.float32), pltpu.VMEM((CH,), i32),
                       pltpu.VMEM((CH,), i32), pltpu.VMEM((H1 // 4,), i32),
                       pltpu.VMEM((NR * 64,), i32), pltpu.VMEM((NR * 64,), i32)])
    return p1, p2, p3, p4


def _unique_pad128(vals):
    sv = jnp.sort(vals, axis=1)
    first = jnp.concatenate(
        [jnp.ones((NG, 1), bool), sv[:, 1:] > sv[:, :-1]], axis=1)
    us = jnp.sort(jnp.where(first, sv, BIG), axis=1)
    cells = jnp.concatenate(
        [us, jnp.full((NG, 128 - 2 * NQ), BIG, jnp.int32)], axis=1)
    slot = _count_lt(cells, vals)
    return cells, slot


def _pack_bytes(b):
    m = b.reshape(-1, 4)
    return m[:, 0] | (m[:, 1] << 8) | (m[:, 2] << 16) | (m[:, 3] << 24)


def _count_lt(rows, q):
    return (rows[:, :, None] < q[:, None, :]).sum(1, dtype=jnp.int32)


def _count_le(rows, q):
    return (rows[:, :, None] <= q[:, None, :]).sum(1, dtype=jnp.int32)


def _digit_tables(bins):
    eb = lax.bitcast_convert_type(bins, jnp.uint32)
    ekeys = jnp.sort(jnp.where(eb >= jnp.uint32(0x80000000),
                               jnp.uint32(0xFFFFFFFF),
                               jnp.uint32(0x80000000)) ^ eb, axis=1)

    b16 = jnp.broadcast_to(jnp.arange(65536, dtype=jnp.uint32)
                           << jnp.uint32(16), (NG, 65536))
    base16 = _count_lt(ekeys, b16)
    cnt16 = _count_le(ekeys, b16 + jnp.uint32(0xFFFF)) - base16
    amb1 = (cnt16 > 0).reshape(-1)
    rs1 = jnp.cumsum(amb1.astype(jnp.int32)) - 1
    l1 = jnp.where(amb1, 51 + jnp.clip(rs1, 0, NR - 8), base16.reshape(-1))

    flat = jnp.arange(NG * 65536, dtype=jnp.int32)
    gb = -lax.top_k(-jnp.where(amb1, flat, BIG - NR), NR)[0]
    gb = jnp.clip(gb, 0, NG * 65536 - 1)
    g1, b1b = gb // 65536, gb % 65536

    mids = jnp.arange(256, dtype=jnp.uint32) << jnp.uint32(8)
    qk2 = (b1b.astype(jnp.uint32) << jnp.uint32(16))[:, None] + mids[None, :]
    ek2 = ekeys[g1]
    base24 = _count_lt(ek2, qk2)
    cnt24 = _count_le(ek2, qk2 + jnp.uint32(0xFF)) - base24
    amb2 = (cnt24 > 0).reshape(-1)
    rs2 = jnp.cumsum(amb2.astype(jnp.int32)) - 1
    l2 = jnp.where(amb2, 51 + jnp.clip(rs2, 0, NR - 8), base24.reshape(-1))

    flat2 = jnp.arange(NR * 256, dtype=jnp.int32)
    sel2 = -lax.top_k(-jnp.where(amb2, flat2, BIG - NR), NR)[0]
    sel2 = jnp.clip(sel2, 0, NR * 256 - 1)
    qk3 = qk2.reshape(-1)[sel2]
    g2 = g1[sel2 // 256]
    lows = jnp.arange(256, dtype=jnp.uint32)
    fk = qk3[:, None] + lows[None, :]
    l3 = _count_le(ekeys[g2], fk).reshape(-1)

    return _pack_bytes(l1), _pack_bytes(l2), _pack_bytes(l3)


def _pack_map(total, idx, valid):
    vals = jnp.where(valid, jnp.arange(1, 129, dtype=jnp.int32)[None, :], 0)
    safe = jnp.where(valid, idx, total).reshape(-1)
    m = jnp.zeros((total + 8,), jnp.int32).at[safe].set(vals.reshape(-1))
    m = m[:total].reshape(-1, 4)
    return m[:, 0] | (m[:, 1] << 8) | (m[:, 2] << 16) | (m[:, 3] << 24)


def _rank_step(csum_rows, ranks):
    nb = csum_rows.shape[1]
    b = jnp.clip(_count_le(csum_rows, ranks), 0, nb - 1)
    prev = jnp.take_along_axis(csum_rows, jnp.clip(b - 1, 0, nb - 1), axis=1)
    prev = jnp.where(b > 0, prev, 0)
    return b, ranks - prev


def kernel(x, group):
    p1, p2, p3, p4 = _kernels()

    h1 = p1(x, group)
    h1 = h1.reshape(NC, H1P)[:, :H1].sum(0).reshape(NG, 65536)
    csum1 = jnp.cumsum(h1, axis=1)
    n_g = csum1[:, -1]

    qs = jnp.linspace(0.0, 1.0, NBINS - 1)
    pos = qs[None, :] * jnp.maximum(n_g - 1, 0).astype(jnp.float32)[:, None]
    lo = jnp.clip(jnp.floor(pos).astype(jnp.int32), 0, N - 1)
    hi = jnp.clip(jnp.ceil(pos).astype(jnp.int32), 0, N - 1)
    frac = pos - jnp.floor(pos)
    ranks = jnp.stack([lo, hi], axis=-1).reshape(NG, 2 * NQ)

    b1, r1 = _rank_step(csum1, ranks)
    cells2, slot2 = _unique_pad128(b1)
    gslot = jnp.arange(NG, dtype=jnp.int32)[:, None] * 128 + slot2
    m16 = _pack_map(H1, jnp.arange(NG, dtype=jnp.int32)[:, None] * 65536 + cells2,
                    cells2 < BIG)

    h2 = p2(x, group, m16)
    h2 = h2.reshape(NC, H2P)[:, :H2].sum(0).reshape(S2, 256)
    csum2 = jnp.cumsum(h2, axis=1)
    rows2 = csum2[gslot.reshape(-1)]
    b2, r2 = _rank_step(rows2, r1.reshape(-1)[:, None])
    b2, r2 = b2[:, 0], r2[:, 0]

    cell24 = (gslot.reshape(-1) * 256 + b2).reshape(NG, 2 * NQ)
    cells3, slot3 = _unique_pad128(cell24)
    gslot3 = jnp.arange(NG, dtype=jnp.int32)[:, None] * 128 + slot3
    m24 = _pack_map(H2, cells3, cells3 < BIG)

    h3 = p3(x, group, m16, m24)
    h3 = h3.reshape(NC, H2P)[:, :H2].sum(0).reshape(S2, 256)
    csum3 = jnp.cumsum(h3, axis=1)
    rows3 = csum3[gslot3.reshape(-1)]
    b3, _ = _rank_step(rows3, r2[:, None])
    b3 = b3[:, 0]

    keyfull = ((b1.reshape(-1).astype(jnp.uint32) << jnp.uint32(16))
               | (b2.astype(jnp.uint32) << jnp.uint32(8))
               | b3.astype(jnp.uint32))
    fb = jnp.where(keyfull >= jnp.uint32(0x80000000),
                   keyfull ^ jnp.uint32(0x80000000), ~keyfull)
    svals = lax.bitcast_convert_type(fb, jnp.float32).reshape(NG, NQ, 2)
    s_lo, s_hi = svals[:, :, 0], svals[:, :, 1]
    bins = s_lo * (1.0 - frac) + s_hi * frac
    l1, l2, l3 = _digit_tables(bins)
    return p4(x, group, l1, l2, l3)


# speedup vs baseline: 28.2817x; 1.1453x over previous
import functools

import jax
import jax.numpy as jnp
from jax import lax
from jax.experimental import pallas as pl
from jax.experimental.pallas import tpu as pltpu
from jax.experimental.pallas import tpu_sc as plsc

N = 8_000_000
NG = 4
NBINS = 51
NQ = NBINS - 1
NC = 2
NS = 16
NW = NC * NS
PERW = N // NW
CH = 10_000
NCHUNK = PERW // CH
CH3 = 5_952
NCHUNK3 = PERW // CH3
CH3T = PERW - NCHUNK3 * CH3

H1 = NG * 65536
H1P = H1 + 128
H1S = H1P // NS
S2 = NG * 128
H2 = S2 * 256
H2P = H2 + 128
H2S = H2P // NS
BIG = 2**31 - 1

@functools.cache
def _mesh():
    return plsc.VectorSubcoreMesh(
        core_axis_name="c", subcore_axis_name="s",
        num_cores=NC, num_subcores=NS)


def _key_u32(xk):
    b = lax.bitcast_convert_type(xk, jnp.uint32)
    flip = jnp.where(b >= jnp.uint32(0x80000000),
                     jnp.uint32(0xFFFFFFFF), jnp.uint32(0x80000000))
    return b ^ flip


def _fill(ref, nelem, value, dtype):
    def body(i, _):
        ref[pl.ds(i * 16, 16)] = jnp.full((16,), value, dtype)
        return 0
    lax.fori_loop(0, nelem // 16, body, 0)


def _byte_lookup(words_ref, cell):
    w = plsc.load_gather(words_ref, [cell >> 2])
    return (w >> ((cell & 3) * 8)) & 255


def _p1_body(x_hbm, g_hbm, out_hbm, xv, gv, idxv, onesv, spm):
    c = lax.axis_index("c")
    s = lax.axis_index("s")
    wid = s * NC + c
    dump = H1 + s * 8
    _fill(idxv, CH, 0, jnp.int32)
    pltpu.sync_copy(idxv, spm.at[pl.ds(s * H1S, CH)])
    pltpu.sync_copy(idxv.at[pl.ds(0, H1S - CH)],
                    spm.at[pl.ds(s * H1S + CH, H1S - CH)])
    _fill(onesv, CH, 1, jnp.int32)
    plsc.subcore_barrier()

    def chunk(j, _):
        base = wid * PERW + j * CH
        pltpu.sync_copy(x_hbm.at[pl.ds(base, CH)], xv)
        pltpu.sync_copy(g_hbm.at[pl.ds(base, CH)], gv)

        @plsc.parallel_loop(0, CH // 16, unroll=4)
        def inner(i):
            xk = xv[pl.ds(i * 16, 16)]
            gk = gv[pl.ds(i * 16, 16)]
            key = _key_u32(xk)
            cell = gk * 65536 + (key >> jnp.uint32(16)).astype(jnp.int32)
            cell = jnp.where(xk == 0.0, dump, cell)
            idxv[pl.ds(i * 16, 16)] = cell
        pltpu.sync_copy(onesv, spm.at[idxv], add=True)
        return 0
    lax.fori_loop(0, NCHUNK, chunk, 0)
    plsc.subcore_barrier()
    for off, sz in ((0, 8192), (8192, H1S - 8192)):
        pltpu.sync_copy(spm.at[pl.ds(s * H1S + off, sz)], idxv.at[pl.ds(0, sz)])
        pltpu.sync_copy(idxv.at[pl.ds(0, sz)],
                        out_hbm.at[pl.ds(c * H1P + s * H1S + off, sz)])


def _p2_body(x_hbm, g_hbm, m16_hbm, out_hbm, xv, gv, idxv, onesv, m16v, spm):
    c = lax.axis_index("c")
    s = lax.axis_index("s")
    wid = s * NC + c
    dump = H2 + s * 8
    pltpu.sync_copy(m16_hbm, m16v)
    _fill(idxv, CH, 0, jnp.int32)
    pltpu.sync_copy(idxv.at[pl.ds(0, H2S)], spm.at[pl.ds(s * H2S, H2S)])
    _fill(onesv, CH, 1, jnp.int32)
    plsc.subcore_barrier()

    def chunk(j, _):
        base = wid * PERW + j * CH
        pltpu.sync_copy(x_hbm.at[pl.ds(base, CH)], xv)
        pltpu.sync_copy(g_hbm.at[pl.ds(base, CH)], gv)

        @plsc.parallel_loop(0, CH // 16, unroll=4)
        def inner(i):
            xk = xv[pl.ds(i * 16, 16)]
            gk = gv[pl.ds(i * 16, 16)]
            key = _key_u32(xk)
            c16 = gk * 65536 + (key >> jnp.uint32(16)).astype(jnp.int32)
            v = _byte_lookup(m16v, c16)
            valid = (xk != 0.0) & (v > 0)
            gslot = gk * 128 + v - 1
            cell = gslot * 256 + ((key >> jnp.uint32(8)).astype(jnp.int32) & 255)
            idxv[pl.ds(i * 16, 16)] = jnp.where(valid, cell, dump)
        pltpu.sync_copy(onesv, spm.at[idxv], add=True)
        return 0
    lax.fori_loop(0, NCHUNK, chunk, 0)
    plsc.subcore_barrier()
    pltpu.sync_copy(spm.at[pl.ds(s * H2S, H2S)], idxv.at[pl.ds(0, H2S)])
    pltpu.sync_copy(idxv.at[pl.ds(0, H2S)],
                    out_hbm.at[pl.ds(c * H2P + s * H2S, H2S)])


def _p3_body(x_hbm, g_hbm, m16_hbm, m24_hbm, out_hbm,
             xv, gv, idxv, onesv, m16v, m24v, spm):
    c = lax.axis_index("c")
    s = lax.axis_index("s")
    wid = s * NC + c
    dump = H2 + s * 8
    pltpu.sync_copy(m16_hbm, m16v)
    pltpu.sync_copy(m24_hbm, m24v)
    _fill(idxv, CH3, 0, jnp.int32)
    for q in range(5):
        pltpu.sync_copy(idxv.at[pl.ds(0, H2S // 5)],
                        spm.at[pl.ds(s * H2S + q * (H2S // 5), H2S // 5)])
    _fill(onesv, CH3, 1, jnp.int32)
    plsc.subcore_barrier()

    def cells_for(i):
        xk = xv[pl.ds(i * 16, 16)]
        gk = gv[pl.ds(i * 16, 16)]
        key = _key_u32(xk)
        c16 = gk * 65536 + (key >> jnp.uint32(16)).astype(jnp.int32)
        v = _byte_lookup(m16v, c16)
        valid = (xk != 0.0) & (v > 0)
        gslot = gk * 128 + v - 1
        c24 = gslot * 256 + ((key >> jnp.uint32(8)).astype(jnp.int32) & 255)
        v3 = _byte_lookup(m24v, jnp.where(valid, c24, 0))
        valid = valid & (v3 > 0)
        gslot3 = gk * 128 + v3 - 1
        cell = gslot3 * 256 + (key.astype(jnp.int32) & 255)
        idxv[pl.ds(i * 16, 16)] = jnp.where(valid, cell, dump)

    def chunk(j, _):
        base = wid * PERW + j * CH3
        pltpu.sync_copy(x_hbm.at[pl.ds(base, CH3)], xv)
        pltpu.sync_copy(g_hbm.at[pl.ds(base, CH3)], gv)

        @plsc.parallel_loop(0, CH3 // 16, unroll=4)
        def inner(i):
            cells_for(i)
        pltpu.sync_copy(onesv, spm.at[idxv], add=True)
        return 0
    lax.fori_loop(0, NCHUNK3, chunk, 0)

    @plsc.parallel_loop(0, CH3 // 16, unroll=4)
    def pad(i):
        idxv[pl.ds(i * 16, 16)] = jnp.full((16,), 0, jnp.int32) + dump
    tbase = wid * PERW + NCHUNK3 * CH3
    pltpu.sync_copy(x_hbm.at[pl.ds(tbase, CH3T)], xv.at[pl.ds(0, CH3T)])
    pltpu.sync_copy(g_hbm.at[pl.ds(tbase, CH3T)], gv.at[pl.ds(0, CH3T)])

    @plsc.parallel_loop(0, CH3T // 16, unroll=1)
    def tail(i):
        cells_for(i)
    pltpu.sync_copy(onesv, spm.at[idxv], add=True)
    plsc.subcore_barrier()
    for q in range(5):
        pltpu.sync_copy(spm.at[pl.ds(s * H2S + q * (H2S // 5), H2S // 5)],
                        idxv.at[pl.ds(0, H2S // 5)])
        pltpu.sync_copy(idxv.at[pl.ds(0, H2S // 5)],
                        out_hbm.at[pl.ds(c * H2P + s * H2S + q * (H2S // 5),
                                         H2S // 5)])


NR = 208


def _p4_body(x_hbm, g_hbm, l1_hbm, l2_hbm, l3_hbm, out_hbm,
             xv, gv, outv, l1v, l2v, l3v):
    c = lax.axis_index("c")
    s = lax.axis_index("s")
    wid = s * NC + c
    pltpu.sync_copy(l1_hbm, l1v)
    pltpu.sync_copy(l2_hbm, l2v)
    pltpu.sync_copy(l3_hbm, l3v)

    def chunk(j, _):
        base = wid * PERW + j * CH
        pltpu.sync_copy(x_hbm.at[pl.ds(base, CH)], xv)
        pltpu.sync_copy(g_hbm.at[pl.ds(base, CH)], gv)

        @plsc.parallel_loop(0, CH // 16, unroll=4)
        def inner(i):
            xk = xv[pl.ds(i * 16, 16)]
            gk = gv[pl.ds(i * 16, 16)]
            key = _key_u32(xk)
            c14 = gk * 16384 + (key >> jnp.uint32(18)).astype(jnp.int32)
            v1 = _byte_lookup(l1v, c14)
            mid = (key >> jnp.uint32(9)).astype(jnp.int32) & 511
            t2 = jnp.where(v1 > 50, (v1 - 51) * 512 + mid, 0)
            v2 = _byte_lookup(l2v, t2)
            low = key.astype(jnp.int32) & 511
            t3 = jnp.where(v2 > 50, (v2 - 51) * 512 + low, 0)
            v3 = _byte_lookup(l3v, t3)
            d = jnp.where(v1 <= 50, v1, jnp.where(v2 <= 50, v2, v3))
            outv[pl.ds(i * 16, 16)] = jnp.where(xk != 0.0, d, 0)
        pltpu.sync_copy(outv, out_hbm.at[pl.ds(base, CH)])
        return 0
    lax.fori_loop(0, NCHUNK, chunk, 0)


@functools.cache
def _kernels():
    i32 = jnp.int32
    cp = pltpu.CompilerParams(needs_layout_passes=False)
    p1 = pl.kernel(
        _p1_body,
        out_type=jax.ShapeDtypeStruct((NC * H1P,), i32),
        mesh=_mesh(),
        compiler_params=cp,
        scratch_types=[pltpu.VMEM((CH,), jnp.float32), pltpu.VMEM((CH,), i32),
                       pltpu.VMEM((CH,), i32), pltpu.VMEM((CH,), i32),
                       pltpu.VMEM_SHARED((H1P,), i32)])
    p2 = pl.kernel(
        _p2_body,
        out_type=jax.ShapeDtypeStruct((NC * H2P,), i32),
        mesh=_mesh(),
        compiler_params=cp,
        scratch_types=[pltpu.VMEM((CH,), jnp.float32), pltpu.VMEM((CH,), i32),
                       pltpu.VMEM((CH,), i32), pltpu.VMEM((CH,), i32),
                       pltpu.VMEM((H1 // 4,), i32),
                       pltpu.VMEM_SHARED((H2P,), i32)])
    p3 = pl.kernel(
        _p3_body,
        out_type=jax.ShapeDtypeStruct((NC * H2P,), i32),
        mesh=_mesh(),
        compiler_params=cp,
        scratch_types=[pltpu.VMEM((CH3,), jnp.float32), pltpu.VMEM((CH3,), i32),
                       pltpu.VMEM((CH3,), i32), pltpu.VMEM((CH3,), i32),
                       pltpu.VMEM((H1 // 4,), i32), pltpu.VMEM((H2 // 4,), i32),
                       pltpu.VMEM_SHARED((H2P,), i32)])
    p4 = pl.kernel(
        _p4_body,
        out_type=jax.ShapeDtypeStruct((N,), i32),
        mesh=_mesh(),
        compiler_params=cp,
        scratch_types=[pltpu.VMEM((CH,), jnp.float32), pltpu.VMEM((CH,), i32),
                       pltpu.VMEM((CH,), i32), pltpu.VMEM((NG * 4096,), i32),
                       pltpu.VMEM((NR * 128,), i32),
                       pltpu.VMEM((NR * 128,), i32)])
    return p1, p2, p3, p4


def _unique_pad128(vals):
    sv = jnp.sort(vals, axis=1)
    first = jnp.concatenate(
        [jnp.ones((NG, 1), bool), sv[:, 1:] > sv[:, :-1]], axis=1)
    us = jnp.sort(jnp.where(first, sv, BIG), axis=1)
    cells = jnp.concatenate(
        [us, jnp.full((NG, 128 - 2 * NQ), BIG, jnp.int32)], axis=1)
    slot = _count_lt(cells, vals)
    return cells, slot


def _pack_bytes(b):
    m = b.reshape(-1, 4)
    return m[:, 0] | (m[:, 1] << 8) | (m[:, 2] << 16) | (m[:, 3] << 24)


def _count_lt(rows, q):
    return (rows[:, :, None] < q[:, None, :]).sum(1, dtype=jnp.int32)


def _count_le(rows, q):
    return (rows[:, :, None] <= q[:, None, :]).sum(1, dtype=jnp.int32)


def _digit_tables(bins):
    eb = lax.bitcast_convert_type(bins, jnp.uint32)
    ekeys = jnp.sort(jnp.where(eb >= jnp.uint32(0x80000000),
                               jnp.uint32(0xFFFFFFFF),
                               jnp.uint32(0x80000000)) ^ eb, axis=1)

    b14 = jnp.broadcast_to(jnp.arange(16384, dtype=jnp.uint32)
                           << jnp.uint32(18), (NG, 16384))
    base1 = _count_lt(ekeys, b14)
    cnt1 = _count_le(ekeys, b14 + jnp.uint32(0x3FFFF)) - base1
    amb1 = (cnt1 > 0).reshape(-1)
    rs1 = jnp.cumsum(amb1.astype(jnp.int32)) - 1
    l1 = jnp.where(amb1, 51 + jnp.clip(rs1, 0, NR - 8), base1.reshape(-1))

    flat = jnp.arange(NG * 16384, dtype=jnp.int32)
    gb = -lax.top_k(-jnp.where(amb1, flat, BIG - NR), NR)[0]
    gb = jnp.clip(gb, 0, NG * 16384 - 1)
    g1, b1b = gb // 16384, gb % 16384

    mids = jnp.arange(512, dtype=jnp.uint32) << jnp.uint32(9)
    qk2 = (b1b.astype(jnp.uint32) << jnp.uint32(18))[:, None] + mids[None, :]
    ek2 = ekeys[g1]
    base2 = _count_lt(ek2, qk2)
    cnt2 = _count_le(ek2, qk2 + jnp.uint32(0x1FF)) - base2
    amb2 = (cnt2 > 0).reshape(-1)
    rs2 = jnp.cumsum(amb2.astype(jnp.int32)) - 1
    l2 = jnp.where(amb2, 51 + jnp.clip(rs2, 0, NR - 8), base2.reshape(-1))

    flat2 = jnp.arange(NR * 512, dtype=jnp.int32)
    sel2 = -lax.top_k(-jnp.where(amb2, flat2, BIG - NR), NR)[0]
    sel2 = jnp.clip(sel2, 0, NR * 512 - 1)
    qk3 = qk2.reshape(-1)[sel2]
    g2 = g1[sel2 // 512]
    lows = jnp.arange(512, dtype=jnp.uint32)
    fk = qk3[:, None] + lows[None, :]
    l3 = _count_le(ekeys[g2], fk).reshape(-1)

    return _pack_bytes(l1), _pack_bytes(l2), _pack_bytes(l3)


def _pack_map(total, idx, valid):
    vals = jnp.where(valid, jnp.arange(1, 129, dtype=jnp.int32)[None, :], 0)
    safe = jnp.where(valid, idx, total).reshape(-1)
    m = jnp.zeros((total + 8,), jnp.int32).at[safe].set(vals.reshape(-1))
    m = m[:total].reshape(-1, 4)
    return m[:, 0] | (m[:, 1] << 8) | (m[:, 2] << 16) | (m[:, 3] << 24)


def _rank_step(csum_rows, ranks):
    nb = csum_rows.shape[1]
    b = jnp.clip(_count_le(csum_rows, ranks), 0, nb - 1)
    prev = jnp.take_along_axis(csum_rows, jnp.clip(b - 1, 0, nb - 1), axis=1)
    prev = jnp.where(b > 0, prev, 0)
    return b, ranks - prev


def kernel(x, group):
    p1, p2, p3, p4 = _kernels()

    h1 = p1(x, group)
    h1 = h1.reshape(NC, H1P)[:, :H1].sum(0).reshape(NG, 65536)
    csum1 = jnp.cumsum(h1, axis=1)
    n_g = csum1[:, -1]

    qs = jnp.linspace(0.0, 1.0, NBINS - 1)
    pos = qs[None, :] * jnp.maximum(n_g - 1, 0).astype(jnp.float32)[:, None]
    lo = jnp.clip(jnp.floor(pos).astype(jnp.int32), 0, N - 1)
    hi = jnp.clip(jnp.ceil(pos).astype(jnp.int32), 0, N - 1)
    frac = pos - jnp.floor(pos)
    ranks = jnp.stack([lo, hi], axis=-1).reshape(NG, 2 * NQ)

    b1, r1 = _rank_step(csum1, ranks)
    cells2, slot2 = _unique_pad128(b1)
    gslot = jnp.arange(NG, dtype=jnp.int32)[:, None] * 128 + slot2
    m16 = _pack_map(H1, jnp.arange(NG, dtype=jnp.int32)[:, None] * 65536 + cells2,
                    cells2 < BIG)

    h2 = p2(x, group, m16)
    h2 = h2.reshape(NC, H2P)[:, :H2].sum(0).reshape(S2, 256)
    csum2 = jnp.cumsum(h2, axis=1)
    rows2 = csum2[gslot.reshape(-1)]
    b2, r2 = _rank_step(rows2, r1.reshape(-1)[:, None])
    b2, r2 = b2[:, 0], r2[:, 0]

    cell24 = (gslot.reshape(-1) * 256 + b2).reshape(NG, 2 * NQ)
    cells3, slot3 = _unique_pad128(cell24)
    gslot3 = jnp.arange(NG, dtype=jnp.int32)[:, None] * 128 + slot3
    m24 = _pack_map(H2, cells3, cells3 < BIG)

    h3 = p3(x, group, m16, m24)
    h3 = h3.reshape(NC, H2P)[:, :H2].sum(0).reshape(S2, 256)
    csum3 = jnp.cumsum(h3, axis=1)
    rows3 = csum3[gslot3.reshape(-1)]
    b3, _ = _rank_step(rows3, r2[:, None])
    b3 = b3[:, 0]

    keyfull = ((b1.reshape(-1).astype(jnp.uint32) << jnp.uint32(16))
               | (b2.astype(jnp.uint32) << jnp.uint32(8))
               | b3.astype(jnp.uint32))
    fb = jnp.where(keyfull >= jnp.uint32(0x80000000),
                   keyfull ^ jnp.uint32(0x80000000), ~keyfull)
    svals = lax.bitcast_convert_type(fb, jnp.float32).reshape(NG, NQ, 2)
    s_lo, s_hi = svals[:, :, 0], svals[:, :, 1]
    bins = s_lo * (1.0 - frac) + s_hi * frac
    l1, l2, l3 = _digit_tables(bins)
    return p4(x, group, l1, l2, l3)


# speedup vs baseline: 29.4091x; 1.0399x over previous
import functools

import jax
import jax.numpy as jnp
from jax import lax
from jax.experimental import pallas as pl
from jax.experimental.pallas import tpu as pltpu
from jax.experimental.pallas import tpu_sc as plsc

N = 8_000_000
NG = 4
NBINS = 51
NQ = NBINS - 1
NC = 2
NS = 16
NW = NC * NS
PERW = N // NW
CH = 10_000
NCHUNK = PERW // CH
CHH = 5_008

H1 = NG * 16384
H1P = H1 + 128
H1S = H1P // NS
S2 = NG * 128
H2 = S2 * 512
H2P = H2 + 128
H2S = H2P // NS
BIG = 2**31 - 1

@functools.cache
def _mesh():
    return plsc.VectorSubcoreMesh(
        core_axis_name="c", subcore_axis_name="s",
        num_cores=NC, num_subcores=NS)


def _key_u32(xk):
    b = lax.bitcast_convert_type(xk, jnp.uint32)
    flip = jnp.where(b >= jnp.uint32(0x80000000),
                     jnp.uint32(0xFFFFFFFF), jnp.uint32(0x80000000))
    return b ^ flip


def _fill(ref, nelem, value, dtype):
    def body(i, _):
        ref[pl.ds(i * 16, 16)] = jnp.full((16,), value, dtype)
        return 0
    lax.fori_loop(0, nelem // 16, body, 0)


def _byte_lookup(words_ref, cell):
    w = plsc.load_gather(words_ref, [cell >> 2])
    return (w >> ((cell & 3) * 8)) & 255


def _p1_body(x_hbm, g_hbm, out_hbm, xv, gv, idxv, onesv, spm):
    c = lax.axis_index("c")
    s = lax.axis_index("s")
    wid = s * NC + c
    dump = H1 + s * 8
    _fill(idxv, CH, 0, jnp.int32)
    pltpu.sync_copy(idxv.at[pl.ds(0, H1S)], spm.at[pl.ds(s * H1S, H1S)])
    _fill(onesv, CH, 1, jnp.int32)
    plsc.subcore_barrier()

    def chunk(j, _):
        base = wid * PERW + j * CH
        pltpu.sync_copy(x_hbm.at[pl.ds(base, CH)], xv)
        pltpu.sync_copy(g_hbm.at[pl.ds(base, CH)], gv)

        @plsc.parallel_loop(0, CH // 16, unroll=4)
        def inner(i):
            xk = xv[pl.ds(i * 16, 16)]
            gk = gv[pl.ds(i * 16, 16)]
            key = _key_u32(xk)
            cell = gk * 16384 + (key >> jnp.uint32(18)).astype(jnp.int32)
            cell = jnp.where(xk == 0.0, dump, cell)
            idxv[pl.ds(i * 16, 16)] = cell
        pltpu.sync_copy(onesv, spm.at[idxv], add=True)
        return 0
    lax.fori_loop(0, NCHUNK, chunk, 0)
    plsc.subcore_barrier()
    pltpu.sync_copy(spm.at[pl.ds(s * H1S, H1S)], idxv.at[pl.ds(0, H1S)])
    pltpu.sync_copy(idxv.at[pl.ds(0, H1S)],
                    out_hbm.at[pl.ds(c * H1P + s * H1S, H1S)])


def _p2_body(x_hbm, g_hbm, m16_hbm, out_hbm, xv, gv, idxv, onesv, m16v, spm):
    c = lax.axis_index("c")
    s = lax.axis_index("s")
    wid = s * NC + c
    dump = H2 + s * 8
    pltpu.sync_copy(m16_hbm, m16v)
    _fill(idxv, CH, 0, jnp.int32)
    for off, sz in ((0, CH), (CH, H2S - CH)):
        pltpu.sync_copy(idxv.at[pl.ds(0, sz)],
                        spm.at[pl.ds(s * H2S + off, sz)])
    _fill(onesv, CH, 1, jnp.int32)
    plsc.subcore_barrier()

    def chunk(j, _):
        base = wid * PERW + j * CH
        pltpu.sync_copy(x_hbm.at[pl.ds(base, CH)], xv)
        pltpu.sync_copy(g_hbm.at[pl.ds(base, CH)], gv)

        @plsc.parallel_loop(0, CH // 16, unroll=4)
        def inner(i):
            xk = xv[pl.ds(i * 16, 16)]
            gk = gv[pl.ds(i * 16, 16)]
            key = _key_u32(xk)
            c14 = gk * 16384 + (key >> jnp.uint32(18)).astype(jnp.int32)
            v = _byte_lookup(m16v, c14)
            valid = (xk != 0.0) & (v > 0)
            gslot = gk * 128 + v - 1
            cell = gslot * 512 + ((key >> jnp.uint32(9)).astype(jnp.int32) & 511)
            idxv[pl.ds(i * 16, 16)] = jnp.where(valid, cell, dump)
        pltpu.sync_copy(onesv, spm.at[idxv], add=True)
        return 0
    lax.fori_loop(0, NCHUNK, chunk, 0)
    plsc.subcore_barrier()
    for off, sz in ((0, 8192), (8192, H2S - 8192)):
        pltpu.sync_copy(spm.at[pl.ds(s * H2S + off, sz)], idxv.at[pl.ds(0, sz)])
        pltpu.sync_copy(idxv.at[pl.ds(0, sz)],
                        out_hbm.at[pl.ds(c * H2P + s * H2S + off, sz)])


def _p3_body(x_hbm, g_hbm, m16_hbm, m24_hbm, out_hbm,
             xv, gv, idxv, onesv, m16v, m24v, spm):
    c = lax.axis_index("c")
    s = lax.axis_index("s")
    wid = s * NC + c
    dump = H2 + s * 8
    pieces = ((0, CHH), (CHH, CHH), (2 * CHH, CHH), (3 * CHH, H2S - 3 * CHH))
    pltpu.sync_copy(m16_hbm, m16v)
    pltpu.sync_copy(m24_hbm, m24v)
    _fill(idxv, CHH, 0, jnp.int32)
    for off, sz in pieces:
        pltpu.sync_copy(idxv.at[pl.ds(0, sz)],
                        spm.at[pl.ds(s * H2S + off, sz)])
    _fill(onesv, CHH, 1, jnp.int32)
    plsc.subcore_barrier()

    def chunk(j, _):
        base = wid * PERW + j * CH
        for off, n in ((0, CHH), (CHH, CH - CHH)):
            pltpu.sync_copy(x_hbm.at[pl.ds(base + off, n)], xv.at[pl.ds(0, n)])
            pltpu.sync_copy(g_hbm.at[pl.ds(base + off, n)], gv.at[pl.ds(0, n)])

            @plsc.parallel_loop(0, n // 16, unroll=4)
            def inner(i):
                xk = xv[pl.ds(i * 16, 16)]
                gk = gv[pl.ds(i * 16, 16)]
                key = _key_u32(xk)
                c14 = gk * 16384 + (key >> jnp.uint32(18)).astype(jnp.int32)
                v = _byte_lookup(m16v, c14)
                valid = (xk != 0.0) & (v > 0)
                gslot = gk * 128 + v - 1
                c23 = gslot * 512 + ((key >> jnp.uint32(9)).astype(jnp.int32)
                                     & 511)
                v3 = _byte_lookup(m24v, jnp.where(valid, c23, 0))
                valid = valid & (v3 > 0)
                gslot3 = gk * 128 + v3 - 1
                cell = gslot3 * 512 + (key.astype(jnp.int32) & 511)
                idxv[pl.ds(i * 16, 16)] = jnp.where(valid, cell, dump)
            if n < CHH:
                idxv[pl.ds(CH - CHH, 16)] = jnp.zeros((16,), jnp.int32) + dump
            pltpu.sync_copy(onesv, spm.at[idxv], add=True)
        return 0
    lax.fori_loop(0, NCHUNK, chunk, 0)
    plsc.subcore_barrier()
    for off, sz in pieces:
        pltpu.sync_copy(spm.at[pl.ds(s * H2S + off, sz)], idxv.at[pl.ds(0, sz)])
        pltpu.sync_copy(idxv.at[pl.ds(0, sz)],
                        out_hbm.at[pl.ds(c * H2P + s * H2S + off, sz)])


NR = 208


def _p4_body(x_hbm, g_hbm, l1_hbm, l2_hbm, l3_hbm, out_hbm,
             xv, gv, outv, l1v, l2v, l3v):
    c = lax.axis_index("c")
    s = lax.axis_index("s")
    wid = s * NC + c
    pltpu.sync_copy(l1_hbm, l1v)
    pltpu.sync_copy(l2_hbm, l2v)
    pltpu.sync_copy(l3_hbm, l3v)

    def chunk(j, _):
        base = wid * PERW + j * CH
        pltpu.sync_copy(x_hbm.at[pl.ds(base, CH)], xv)
        pltpu.sync_copy(g_hbm.at[pl.ds(base, CH)], gv)

        @plsc.parallel_loop(0, CH // 16, unroll=4)
        def inner(i):
            xk = xv[pl.ds(i * 16, 16)]
            gk = gv[pl.ds(i * 16, 16)]
            key = _key_u32(xk)
            c14 = gk * 16384 + (key >> jnp.uint32(18)).astype(jnp.int32)
            v1 = _byte_lookup(l1v, c14)
            mid = (key >> jnp.uint32(9)).astype(jnp.int32) & 511
            t2 = jnp.where(v1 > 50, (v1 - 51) * 512 + mid, 0)
            v2 = _byte_lookup(l2v, t2)
            low = key.astype(jnp.int32) & 511
            t3 = jnp.where(v2 > 50, (v2 - 51) * 512 + low, 0)
            v3 = _byte_lookup(l3v, t3)
            d = jnp.where(v1 <= 50, v1, jnp.where(v2 <= 50, v2, v3))
            outv[pl.ds(i * 16, 16)] = jnp.where(xk != 0.0, d, 0)
        pltpu.sync_copy(outv, out_hbm.at[pl.ds(base, CH)])
        return 0
    lax.fori_loop(0, NCHUNK, chunk, 0)


@functools.cache
def _kernels():
    i32 = jnp.int32
    cp = pltpu.CompilerParams(needs_layout_passes=False)
    p1 = pl.kernel(
        _p1_body,
        out_type=jax.ShapeDtypeStruct((NC * H1P,), i32),
        mesh=_mesh(),
        compiler_params=cp,
        scratch_types=[pltpu.VMEM((CH,), jnp.float32), pltpu.VMEM((CH,), i32),
                       pltpu.VMEM((CH,), i32), pltpu.VMEM((CH,), i32),
                       pltpu.VMEM_SHARED((H1P,), i32)])
    p2 = pl.kernel(
        _p2_body,
        out_type=jax.ShapeDtypeStruct((NC * H2P,), i32),
        mesh=_mesh(),
        compiler_params=cp,
        scratch_types=[pltpu.VMEM((CH,), jnp.float32), pltpu.VMEM((CH,), i32),
                       pltpu.VMEM((CH,), i32), pltpu.VMEM((CH,), i32),
                       pltpu.VMEM((H1 // 4,), i32),
                       pltpu.VMEM_SHARED((H2P,), i32)])
    p3 = pl.kernel(
        _p3_body,
        out_type=jax.ShapeDtypeStruct((NC * H2P,), i32),
        mesh=_mesh(),
        compiler_params=cp,
        scratch_types=[pltpu.VMEM((CHH,), jnp.float32), pltpu.VMEM((CHH,), i32),
                       pltpu.VMEM((CHH,), i32), pltpu.VMEM((CHH,), i32),
                       pltpu.VMEM((H1 // 4,), i32), pltpu.VMEM((H2 // 4,), i32),
                       pltpu.VMEM_SHARED((H2P,), i32)])
    p4 = pl.kernel(
        _p4_body,
        out_type=jax.ShapeDtypeStruct((N,), i32),
        mesh=_mesh(),
        compiler_params=cp,
        scratch_types=[pltpu.VMEM((CH,), jnp.float32), pltpu.VMEM((CH,), i32),
                       pltpu.VMEM((CH,), i32), pltpu.VMEM((NG * 4096,), i32),
                       pltpu.VMEM((NR * 128,), i32),
                       pltpu.VMEM((NR * 128,), i32)])
    return p1, p2, p3, p4


def _unique_pad128(vals):
    sv = jnp.sort(vals, axis=1)
    first = jnp.concatenate(
        [jnp.ones((NG, 1), bool), sv[:, 1:] > sv[:, :-1]], axis=1)
    us = jnp.sort(jnp.where(first, sv, BIG), axis=1)
    cells = jnp.concatenate(
        [us, jnp.full((NG, 128 - 2 * NQ), BIG, jnp.int32)], axis=1)
    slot = _count_lt(cells, vals)
    return cells, slot


def _pack_bytes(b):
    m = b.reshape(-1, 4)
    return m[:, 0] | (m[:, 1] << 8) | (m[:, 2] << 16) | (m[:, 3] << 24)


def _count_lt(rows, q):
    return (rows[:, :, None] < q[:, None, :]).sum(1, dtype=jnp.int32)


def _count_le(rows, q):
    return (rows[:, :, None] <= q[:, None, :]).sum(1, dtype=jnp.int32)


def _digit_tables(bins):
    eb = lax.bitcast_convert_type(bins, jnp.uint32)
    ekeys = jnp.sort(jnp.where(eb >= jnp.uint32(0x80000000),
                               jnp.uint32(0xFFFFFFFF),
                               jnp.uint32(0x80000000)) ^ eb, axis=1)

    b14 = jnp.broadcast_to(jnp.arange(16384, dtype=jnp.uint32)
                           << jnp.uint32(18), (NG, 16384))
    base1 = _count_lt(ekeys, b14)
    cnt1 = _count_le(ekeys, b14 + jnp.uint32(0x3FFFF)) - base1
    amb1 = (cnt1 > 0).reshape(-1)
    rs1 = jnp.cumsum(amb1.astype(jnp.int32)) - 1
    l1 = jnp.where(amb1, 51 + jnp.clip(rs1, 0, NR - 8), base1.reshape(-1))

    flat = jnp.arange(NG * 16384, dtype=jnp.int32)
    gb = -lax.top_k(-jnp.where(amb1, flat, BIG - NR), NR)[0]
    gb = jnp.clip(gb, 0, NG * 16384 - 1)
    g1, b1b = gb // 16384, gb % 16384

    mids = jnp.arange(512, dtype=jnp.uint32) << jnp.uint32(9)
    qk2 = (b1b.astype(jnp.uint32) << jnp.uint32(18))[:, None] + mids[None, :]
    ek2 = ekeys[g1]
    base2 = _count_lt(ek2, qk2)
    cnt2 = _count_le(ek2, qk2 + jnp.uint32(0x1FF)) - base2
    amb2 = (cnt2 > 0).reshape(-1)
    rs2 = jnp.cumsum(amb2.astype(jnp.int32)) - 1
    l2 = jnp.where(amb2, 51 + jnp.clip(rs2, 0, NR - 8), base2.reshape(-1))

    flat2 = jnp.arange(NR * 512, dtype=jnp.int32)
    sel2 = -lax.top_k(-jnp.where(amb2, flat2, BIG - NR), NR)[0]
    sel2 = jnp.clip(sel2, 0, NR * 512 - 1)
    qk3 = qk2.reshape(-1)[sel2]
    g2 = g1[sel2 // 512]
    lows = jnp.arange(512, dtype=jnp.uint32)
    fk = qk3[:, None] + lows[None, :]
    l3 = _count_le(ekeys[g2], fk).reshape(-1)

    return _pack_bytes(l1), _pack_bytes(l2), _pack_bytes(l3)


def _pack_map(total, idx, valid):
    vals = jnp.where(valid, jnp.arange(1, 129, dtype=jnp.int32)[None, :], 0)
    safe = jnp.where(valid, idx, total).reshape(-1)
    m = jnp.zeros((total + 8,), jnp.int32).at[safe].set(vals.reshape(-1))
    m = m[:total].reshape(-1, 4)
    return m[:, 0] | (m[:, 1] << 8) | (m[:, 2] << 16) | (m[:, 3] << 24)


def _rank_step(csum_rows, ranks):
    nb = csum_rows.shape[1]
    b = jnp.clip(_count_le(csum_rows, ranks), 0, nb - 1)
    prev = jnp.take_along_axis(csum_rows, jnp.clip(b - 1, 0, nb - 1), axis=1)
    prev = jnp.where(b > 0, prev, 0)
    return b, ranks - prev


def kernel(x, group):
    p1, p2, p3, p4 = _kernels()

    h1 = p1(x, group)
    h1 = h1.reshape(NC, H1P)[:, :H1].sum(0).reshape(NG, H1 // NG)
    csum1 = jnp.cumsum(h1, axis=1)
    n_g = csum1[:, -1]

    qs = jnp.linspace(0.0, 1.0, NBINS - 1)
    pos = qs[None, :] * jnp.maximum(n_g - 1, 0).astype(jnp.float32)[:, None]
    lo = jnp.clip(jnp.floor(pos).astype(jnp.int32), 0, N - 1)
    hi = jnp.clip(jnp.ceil(pos).astype(jnp.int32), 0, N - 1)
    frac = pos - jnp.floor(pos)
    ranks = jnp.stack([lo, hi], axis=-1).reshape(NG, 2 * NQ)

    b1, r1 = _rank_step(csum1, ranks)
    cells2, slot2 = _unique_pad128(b1)
    gslot = jnp.arange(NG, dtype=jnp.int32)[:, None] * 128 + slot2
    m16 = _pack_map(H1, jnp.arange(NG, dtype=jnp.int32)[:, None] * 16384 + cells2,
                    cells2 < BIG)

    h2 = p2(x, group, m16)
    h2 = h2.reshape(NC, H2P)[:, :H2].sum(0).reshape(S2, 512)
    csum2 = jnp.cumsum(h2, axis=1)
    rows2 = csum2[gslot.reshape(-1)]
    b2, r2 = _rank_step(rows2, r1.reshape(-1)[:, None])
    b2, r2 = b2[:, 0], r2[:, 0]

    cell24 = (gslot.reshape(-1) * 512 + b2).reshape(NG, 2 * NQ)
    cells3, slot3 = _unique_pad128(cell24)
    gslot3 = jnp.arange(NG, dtype=jnp.int32)[:, None] * 128 + slot3
    m24 = _pack_map(H2, cells3, cells3 < BIG)

    h3 = p3(x, group, m16, m24)
    h3 = h3.reshape(NC, H2P)[:, :H2].sum(0).reshape(S2, 512)
    csum3 = jnp.cumsum(h3, axis=1)
    rows3 = csum3[gslot3.reshape(-1)]
    b3, _ = _rank_step(rows3, r2[:, None])
    b3 = b3[:, 0]

    keyfull = ((b1.reshape(-1).astype(jnp.uint32) << jnp.uint32(18))
               | (b2.astype(jnp.uint32) << jnp.uint32(9))
               | b3.astype(jnp.uint32))
    fb = jnp.where(keyfull >= jnp.uint32(0x80000000),
                   keyfull ^ jnp.uint32(0x80000000), ~keyfull)
    svals = lax.bitcast_convert_type(fb, jnp.float32).reshape(NG, NQ, 2)
    s_lo, s_hi = svals[:, :, 0], svals[:, :, 1]
    bins = s_lo * (1.0 - frac) + s_hi * frac
    l1, l2, l3 = _digit_tables(bins)
    return p4(x, group, l1, l2, l3)
